# Initial kernel scaffold; baseline (speedup 1.0000x reference)
import jax
import jax.numpy as jnp
from jax.experimental import pallas as pl


def kernel(node_features, lon_idx, lat_idx, edge_index, lon_table, lat_table, W, score_src, score_tgt, skip_W, bias):
    raise NotImplementedError("write your pallas kernel here")


# speedup vs baseline: 22.7617x; 22.7617x over previous
import functools
import jax
import jax.numpy as jnp
from jax import lax
from jax.experimental import pallas as pl
from jax.experimental.pallas import tpu as pltpu
from jax.experimental.pallas import tpu_sc as plsc

N = 50000
E = 800000
NP = 50176
EP = 819200
TRASH = N
NH = 4
F = 16
ROWS_PER_TILE = NP // 16
CHUNK = 128
CHUNKS_PER_TILE = (EP // 16) // CHUNK

_mesh = plsc.VectorSubcoreMesh(core_axis_name="c", subcore_axis_name="s")
_sc_params = pltpu.CompilerParams(use_tc_tiling_on_sc=False,
                                  needs_layout_passes=False)


def _emb_body(lon_t, lat_t, lon_i, lat_i, lon_o, lat_o, idx_v, rows_v, sem):
  c = lax.axis_index("c")
  s = lax.axis_index("s")
  wid = s * 2 + c
  per_w = NP // 32
  n_ch = per_w // 112

  def do_table(tab, idx_hbm, out_hbm):
    def body(k, _):
      base = wid * per_w + k * 112
      pltpu.sync_copy(idx_hbm.at[pl.ds(base, 112)], idx_v)
      pltpu.async_copy(tab.at[idx_v], rows_v, sem).wait()
      pltpu.sync_copy(rows_v, out_hbm.at[pl.ds(base, 112)])
      return 0
    lax.fori_loop(0, n_ch, body, 0)

  do_table(lon_t, lon_i, lon_o)
  do_table(lat_t, lat_i, lat_o)


_emb_call = pl.kernel(
    _emb_body,
    out_type=(jax.ShapeDtypeStruct((NP, 16), jnp.float32),
              jax.ShapeDtypeStruct((NP, 16), jnp.float32)),
    mesh=_mesh,
    scratch_types=[
        pltpu.VMEM((112,), jnp.int32),
        pltpu.VMEM((112, 16), jnp.float32),
        pltpu.SemaphoreType.DMA,
    ],
    compiler_params=_sc_params,
)


def _proj_body(lon_ref, lat_ref, nf_ref, wl, wt, wn, sl, st, sn, mcat, b,
               proj_o, skip_o, scat_o):
  lon = lon_ref[...]
  lat = lat_ref[...]
  nf = nf_ref[...]
  proj = (jnp.dot(lon, wl[...], preferred_element_type=jnp.float32)
          + jnp.dot(lat, wt[...], preferred_element_type=jnp.float32)
          + jnp.dot(nf, wn[...], preferred_element_type=jnp.float32))
  skip = (jnp.dot(lon, sl[...], preferred_element_type=jnp.float32)
          + jnp.dot(lat, st[...], preferred_element_type=jnp.float32)
          + jnp.dot(nf, sn[...], preferred_element_type=jnp.float32))
  proj_o[...] = proj
  skip_o[...] = skip + b[...]
  scat_o[...] = jnp.dot(proj, mcat[...], preferred_element_type=jnp.float32)


def _proj_call(lon, lat, nf, wl, wt, wn, sl, st, sn, mcat, b):
  g = NP // 256
  blk = lambda w: pl.BlockSpec((256, w), lambda i: (i, 0))
  full = lambda a, b_: pl.BlockSpec((a, b_), lambda i: (0, 0))
  return pl.pallas_call(
      _proj_body,
      grid=(g,),
      in_specs=[blk(16), blk(16), blk(128),
                full(16, 64), full(16, 64), full(128, 64),
                full(16, 64), full(16, 64), full(128, 64),
                full(64, 8), full(1, 64)],
      out_specs=[blk(64), blk(64), blk(8)],
      out_shape=[jax.ShapeDtypeStruct((NP, 64), jnp.float32),
                 jax.ShapeDtypeStruct((NP, 64), jnp.float32),
                 jax.ShapeDtypeStruct((NP, 8), jnp.float32)],
  )(lon, lat, nf, wl, wt, wn, sl, st, sn, mcat, b)


ACC_W = 17
PT_W = 20
ZROWS = ROWS_PER_TILE // 16


def _edge_body(src_e, dst_e, ptab,
               acc_o,
               sidx, didx, sidxa, didxa, prow, stw, mv, zb, cb,
               acc_s, sem0, sem2):
  c = lax.axis_index("c")
  s = lax.axis_index("s")
  band = s * ROWS_PER_TILE
  iota = lax.iota(jnp.int32, 16)
  z16 = jnp.zeros((16,), jnp.float32)

  def zrow_body(i, _):
    zb[i, pl.ds(0, 16)] = z16
    zb[i, pl.ds(1, 16)] = z16
    return 0
  lax.fori_loop(0, ZROWS, zrow_body, 0)

  for hp in range(2):
    hoff = (2 * c + hp) * NP

    def zcp_body(k, _):
      pltpu.sync_copy(zb, acc_s.at[pl.ds(band + k * ZROWS, ZROWS)])
      return 0
    lax.fori_loop(0, 16, zcp_body, 0)
    plsc.subcore_barrier()

    def chunk_body(g, _):
      base = (s * CHUNKS_PER_TILE + g) * CHUNK
      pltpu.sync_copy(src_e.at[pl.ds(base, CHUNK)], sidx)
      pltpu.sync_copy(dst_e.at[pl.ds(base, CHUNK)], didx)
      for g2 in range(CHUNK // 16):
        sl_ = pl.ds(g2 * 16, 16)
        sidxa[sl_] = sidx[sl_] + hoff
        didxa[sl_] = didx[sl_] + hoff
      a0 = pltpu.async_copy(ptab.at[sidxa], prow, sem0)
      a2 = pltpu.async_copy(ptab.at[didxa], stw, sem2)
      a0.wait()
      a2.wait()
      c16 = jnp.full((16,), 16, jnp.int32)
      c17 = jnp.full((16,), 17, jnp.int32)
      for g2 in range(CHUNK // 16):
        rows = iota + g2 * 16
        ev = (plsc.load_gather(prow, [rows, c16])
              + plsc.load_gather(stw, [rows, c17]))
        ev = jnp.where(ev > 0, ev, 0.2 * ev)
        w = jnp.exp(ev)
        plsc.store_scatter(mv, [rows, c16], w)
        for f in range(F):
          col = jnp.full((16,), f, jnp.int32)
          p = plsc.load_gather(prow, [rows, col])
          plsc.store_scatter(mv, [rows, col], p * w)
      pltpu.sync_copy(mv, acc_s.at[didx], add=True)
      return 0

    lax.fori_loop(0, CHUNKS_PER_TILE, chunk_body, 0)
    plsc.subcore_barrier()

    def out_body(k, _):
      r = band + k * ZROWS
      pltpu.sync_copy(acc_s.at[pl.ds(r, ZROWS)], cb)
      pltpu.sync_copy(cb, acc_o.at[pl.ds(hoff + r, ZROWS)])
      return 0
    lax.fori_loop(0, 16, out_body, 0)
    plsc.subcore_barrier()


_edge_call = pl.kernel(
    _edge_body,
    out_type=jax.ShapeDtypeStruct((4 * NP, ACC_W), jnp.float32),
    mesh=_mesh,
    scratch_types=[
        pltpu.VMEM((CHUNK,), jnp.int32),
        pltpu.VMEM((CHUNK,), jnp.int32),
        pltpu.VMEM((CHUNK,), jnp.int32),
        pltpu.VMEM((CHUNK,), jnp.int32),
        pltpu.VMEM((CHUNK, PT_W), jnp.float32),
        pltpu.VMEM((CHUNK, PT_W), jnp.float32),
        pltpu.VMEM((CHUNK, ACC_W), jnp.float32),
        pltpu.VMEM((ZROWS, ACC_W), jnp.float32),
        pltpu.VMEM((ZROWS, ACC_W), jnp.float32),
        pltpu.VMEM_SHARED((NP, ACC_W), jnp.float32),
        pltpu.SemaphoreType.DMA,
        pltpu.SemaphoreType.DMA,
    ],
    compiler_params=_sc_params,
)


def _fin_body(agg_ref, den_ref, skip_ref, out_ref):
  x = agg_ref[...] / (den_ref[...] + 1e-16) + skip_ref[...]
  out_ref[...] = jnp.where(x > 0, x, jnp.exp(x) - 1.0)


def _fin_call(agg, den, skip):
  g = NP // 256
  blk = pl.BlockSpec((256, 64), lambda i: (i, 0))
  return pl.pallas_call(
      _fin_body,
      grid=(g,),
      in_specs=[blk, blk, blk],
      out_specs=blk,
      out_shape=jax.ShapeDtypeStruct((NP, 64), jnp.float32),
  )(agg, den, skip)


@jax.jit
def kernel(node_features, lon_idx, lat_idx, edge_index, lon_table, lat_table,
           W, score_src, score_tgt, skip_W, bias):
  padn = NP - N
  lon_ip = jnp.pad(lon_idx, (0, padn))
  lat_ip = jnp.pad(lat_idx, (0, padn))
  nf_p = jnp.pad(node_features, ((0, padn), (0, 0)))

  src_p = jnp.pad(edge_index[0], (0, EP - E), constant_values=TRASH)
  dst_p = jnp.pad(edge_index[1], (0, EP - E), constant_values=TRASH)

  wl, wt, wn = W[:16], W[16:32], W[32:]
  sl_, st_, sn_ = skip_W[:16], skip_W[16:32], skip_W[32:]
  eye = jnp.eye(NH, dtype=jnp.float32)
  msrc = (score_src[:, :, None] * eye[:, None, :]).reshape(NH * F, NH)
  mtgt = (score_tgt[:, :, None] * eye[:, None, :]).reshape(NH * F, NH)
  mcat = jnp.concatenate([msrc, mtgt], axis=1)
  b2 = bias.reshape(1, 64)

  lon_emb, lat_emb = _emb_call(lon_table, lat_table, lon_ip, lat_ip)

  proj, skip, scat = _proj_call(lon_emb, lat_emb, nf_p,
                                wl, wt, wn, sl_, st_, sn_, mcat, b2)

  zp = jnp.zeros((NP, 2), jnp.float32)
  ptab = jnp.concatenate([
      jnp.concatenate([proj[:, hh * F:(hh + 1) * F],
                       scat[:, hh:hh + 1], scat[:, 4 + hh:5 + hh], zp], axis=1)
      for hh in range(NH)], axis=0)
  acc = _edge_call(src_p, dst_p, ptab)

  agg = jnp.concatenate([acc[hh * NP:(hh + 1) * NP, :F] for hh in range(NH)],
                        axis=1)
  den = jnp.concatenate([acc[hh * NP:(hh + 1) * NP, F:F + 1] for hh in range(NH)],
                        axis=1)
  denx = jnp.repeat(den, F, axis=1)
  out = _fin_call(agg, denx, skip)
  return out[:N]


# speedup vs baseline: 24.6123x; 1.0813x over previous
import functools
import jax
import jax.numpy as jnp
from jax import lax
from jax.experimental import pallas as pl
from jax.experimental.pallas import tpu as pltpu
from jax.experimental.pallas import tpu_sc as plsc

N = 50000
E = 800000
NP = 50176
EP = 819200
TRASH = N
NH = 4
F = 16
ROWS_PER_TILE = NP // 16
CHUNK = 128
CHUNKS_PER_TILE = (EP // 16) // CHUNK

_mesh = plsc.VectorSubcoreMesh(core_axis_name="c", subcore_axis_name="s")
_sc_params = pltpu.CompilerParams(use_tc_tiling_on_sc=False,
                                  needs_layout_passes=False)


def _emb_body(lon_t, lat_t, lon_i, lat_i, lon_o, lat_o, idx_v, rows_v, sem):
  c = lax.axis_index("c")
  s = lax.axis_index("s")
  wid = s * 2 + c
  per_w = NP // 32
  n_ch = per_w // 112

  def do_table(tab, idx_hbm, out_hbm):
    def body(k, _):
      base = wid * per_w + k * 112
      pltpu.sync_copy(idx_hbm.at[pl.ds(base, 112)], idx_v)
      pltpu.async_copy(tab.at[idx_v], rows_v, sem).wait()
      pltpu.sync_copy(rows_v, out_hbm.at[pl.ds(base, 112)])
      return 0
    lax.fori_loop(0, n_ch, body, 0)

  do_table(lon_t, lon_i, lon_o)
  do_table(lat_t, lat_i, lat_o)


_emb_call = pl.kernel(
    _emb_body,
    out_type=(jax.ShapeDtypeStruct((NP, 16), jnp.float32),
              jax.ShapeDtypeStruct((NP, 16), jnp.float32)),
    mesh=_mesh,
    scratch_types=[
        pltpu.VMEM((112,), jnp.int32),
        pltpu.VMEM((112, 16), jnp.float32),
        pltpu.SemaphoreType.DMA,
    ],
    compiler_params=_sc_params,
)


def _proj_body(lon_ref, lat_ref, nf_ref, wl, wt, wn, sl, st, sn, mcat, b,
               proj_o, skip_o, scat_o):
  lon = lon_ref[...]
  lat = lat_ref[...]
  nf = nf_ref[...]
  proj = (jnp.dot(lon, wl[...], preferred_element_type=jnp.float32)
          + jnp.dot(lat, wt[...], preferred_element_type=jnp.float32)
          + jnp.dot(nf, wn[...], preferred_element_type=jnp.float32))
  skip = (jnp.dot(lon, sl[...], preferred_element_type=jnp.float32)
          + jnp.dot(lat, st[...], preferred_element_type=jnp.float32)
          + jnp.dot(nf, sn[...], preferred_element_type=jnp.float32))
  proj_o[...] = proj
  skip_o[...] = skip + b[...]
  scat_o[...] = jnp.dot(proj, mcat[...], preferred_element_type=jnp.float32)


def _proj_call(lon, lat, nf, wl, wt, wn, sl, st, sn, mcat, b):
  g = NP // 256
  blk = lambda w: pl.BlockSpec((256, w), lambda i: (i, 0))
  full = lambda a, b_: pl.BlockSpec((a, b_), lambda i: (0, 0))
  return pl.pallas_call(
      _proj_body,
      grid=(g,),
      in_specs=[blk(16), blk(16), blk(128),
                full(16, 64), full(16, 64), full(128, 64),
                full(16, 64), full(16, 64), full(128, 64),
                full(64, 8), full(1, 64)],
      out_specs=[blk(64), blk(64), blk(8)],
      out_shape=[jax.ShapeDtypeStruct((NP, 64), jnp.float32),
                 jax.ShapeDtypeStruct((NP, 64), jnp.float32),
                 jax.ShapeDtypeStruct((NP, 8), jnp.float32)],
  )(lon, lat, nf, wl, wt, wn, sl, st, sn, mcat, b)


ACC_W = 17
PT_W = 20
ZROWS = ROWS_PER_TILE // 16


def _edge_body(src_e, dst_e, ptab,
               acc_o,
               sidx, didx, sidxa, didxa, prow, stw, mv, zb, cb,
               acc_s, sem0, sem2):
  c = lax.axis_index("c")
  s = lax.axis_index("s")
  band = s * ROWS_PER_TILE
  iota = lax.iota(jnp.int32, 16)
  z16 = jnp.zeros((16,), jnp.float32)

  def zrow_body(i, _):
    zb[i, pl.ds(0, 16)] = z16
    zb[i, pl.ds(1, 16)] = z16
    return 0
  lax.fori_loop(0, ZROWS, zrow_body, 0)

  for hp in range(2):
    hoff = (2 * c + hp) * NP

    def zcp_body(k, _):
      pltpu.sync_copy(zb, acc_s.at[pl.ds(band + k * ZROWS, ZROWS)])
      return 0
    lax.fori_loop(0, 16, zcp_body, 0)
    plsc.subcore_barrier()

    def chunk_body(g, _):
      base = (s * CHUNKS_PER_TILE + g) * CHUNK
      ai = pltpu.async_copy(src_e.at[pl.ds(base, CHUNK)], sidx, sem0)
      aj = pltpu.async_copy(dst_e.at[pl.ds(base, CHUNK)], didx, sem2)
      ai.wait()
      aj.wait()
      for g2 in range(CHUNK // 16):
        sl_ = pl.ds(g2 * 16, 16)
        sidxa[sl_] = sidx[sl_] + hoff
        didxa[sl_] = didx[sl_] + hoff
      a0 = pltpu.async_copy(ptab.at[sidxa], prow, sem0)
      a2 = pltpu.async_copy(ptab.at[didxa], stw, sem2)
      a0.wait()
      a2.wait()
      c16 = jnp.full((16,), 16, jnp.int32)
      c17 = jnp.full((16,), 17, jnp.int32)
      for g2 in range(CHUNK // 16):
        rows = iota + g2 * 16
        ev = (plsc.load_gather(prow, [rows, c16])
              + plsc.load_gather(stw, [rows, c17]))
        ev = jnp.where(ev > 0, ev, 0.2 * ev)
        w = jnp.exp(ev)
        plsc.store_scatter(mv, [rows, c16], w)
        for f in range(F):
          col = jnp.full((16,), f, jnp.int32)
          p = plsc.load_gather(prow, [rows, col])
          plsc.store_scatter(mv, [rows, col], p * w)
      pltpu.sync_copy(mv, acc_s.at[didx], add=True)
      return 0

    lax.fori_loop(0, CHUNKS_PER_TILE, chunk_body, 0)
    plsc.subcore_barrier()

    def out_body(k, _):
      r = band + k * ZROWS
      pltpu.sync_copy(acc_s.at[pl.ds(r, ZROWS)], cb)
      pltpu.sync_copy(cb, acc_o.at[pl.ds(hoff + r, ZROWS)])
      return 0
    lax.fori_loop(0, 16, out_body, 0)
    plsc.subcore_barrier()


_edge_call = pl.kernel(
    _edge_body,
    out_type=jax.ShapeDtypeStruct((4 * NP, ACC_W), jnp.float32),
    mesh=_mesh,
    scratch_types=[
        pltpu.VMEM((CHUNK,), jnp.int32),
        pltpu.VMEM((CHUNK,), jnp.int32),
        pltpu.VMEM((CHUNK,), jnp.int32),
        pltpu.VMEM((CHUNK,), jnp.int32),
        pltpu.VMEM((CHUNK, PT_W), jnp.float32),
        pltpu.VMEM((CHUNK, PT_W), jnp.float32),
        pltpu.VMEM((CHUNK, ACC_W), jnp.float32),
        pltpu.VMEM((ZROWS, ACC_W), jnp.float32),
        pltpu.VMEM((ZROWS, ACC_W), jnp.float32),
        pltpu.VMEM_SHARED((NP, ACC_W), jnp.float32),
        pltpu.SemaphoreType.DMA,
        pltpu.SemaphoreType.DMA,
    ],
    compiler_params=_sc_params,
)


def _fin_body(agg_ref, den_ref, skip_ref, out_ref):
  x = agg_ref[...] / (den_ref[...] + 1e-16) + skip_ref[...]
  out_ref[...] = jnp.where(x > 0, x, jnp.exp(x) - 1.0)


def _fin_call(agg, den, skip):
  g = NP // 256
  blk = pl.BlockSpec((256, 64), lambda i: (i, 0))
  return pl.pallas_call(
      _fin_body,
      grid=(g,),
      in_specs=[blk, blk, blk],
      out_specs=blk,
      out_shape=jax.ShapeDtypeStruct((NP, 64), jnp.float32),
  )(agg, den, skip)


@jax.jit
def kernel(node_features, lon_idx, lat_idx, edge_index, lon_table, lat_table,
           W, score_src, score_tgt, skip_W, bias):
  padn = NP - N
  lon_ip = jnp.pad(lon_idx, (0, padn))
  lat_ip = jnp.pad(lat_idx, (0, padn))
  nf_p = jnp.pad(node_features, ((0, padn), (0, 0)))

  src_p = jnp.pad(edge_index[0], (0, EP - E), constant_values=TRASH)
  dst_p = jnp.pad(edge_index[1], (0, EP - E), constant_values=TRASH)

  wl, wt, wn = W[:16], W[16:32], W[32:]
  sl_, st_, sn_ = skip_W[:16], skip_W[16:32], skip_W[32:]
  eye = jnp.eye(NH, dtype=jnp.float32)
  msrc = (score_src[:, :, None] * eye[:, None, :]).reshape(NH * F, NH)
  mtgt = (score_tgt[:, :, None] * eye[:, None, :]).reshape(NH * F, NH)
  mcat = jnp.concatenate([msrc, mtgt], axis=1)
  b2 = bias.reshape(1, 64)

  lon_emb, lat_emb = _emb_call(lon_table, lat_table, lon_ip, lat_ip)

  proj, skip, scat = _proj_call(lon_emb, lat_emb, nf_p,
                                wl, wt, wn, sl_, st_, sn_, mcat, b2)

  zp = jnp.zeros((NP, 2), jnp.float32)
  ptab = jnp.concatenate([
      jnp.concatenate([proj[:, hh * F:(hh + 1) * F],
                       scat[:, hh:hh + 1], scat[:, 4 + hh:5 + hh], zp], axis=1)
      for hh in range(NH)], axis=0)
  acc = _edge_call(src_p, dst_p, ptab)

  agg = jnp.concatenate([acc[hh * NP:(hh + 1) * NP, :F] for hh in range(NH)],
                        axis=1)
  den = jnp.concatenate([acc[hh * NP:(hh + 1) * NP, F:F + 1] for hh in range(NH)],
                        axis=1)
  denx = jnp.repeat(den, F, axis=1)
  out = _fin_call(agg, denx, skip)
  return out[:N]
